# Initial kernel scaffold; baseline (speedup 1.0000x reference)
import jax
import jax.numpy as jnp
from jax.experimental import pallas as pl


def kernel(x, edge_index, W1, b1, W2, b2):
    raise NotImplementedError("write your pallas kernel here")


# speedup vs baseline: 5.2783x; 5.2783x over previous
import functools

import jax
import jax.numpy as jnp
from jax import lax
from jax.experimental import pallas as pl
from jax.experimental.pallas import tpu as pltpu
from jax.experimental.pallas import tpu_sc as plsc

NC = 2
NS = 16
LB = 128


def _mm1_body(x_ref, w_ref, o_ref):
    c = pl.program_id(1)
    w = w_ref[:, pl.ds(c * 128, 128)]
    o_ref[...] = jnp.dot(x_ref[...], w, preferred_element_type=jnp.float32)


def _mm1(x, W1, bm):
    n, f = x.shape
    grid = (n // bm, 2)
    return pl.pallas_call(
        _mm1_body,
        grid=grid,
        in_specs=[
            pl.BlockSpec((bm, f), lambda i, c: (i, 0)),
            pl.BlockSpec((f, 256), lambda i, c: (0, 0)),
        ],
        out_specs=pl.BlockSpec((bm, 128), lambda i, c: (c * (n // bm) + i, 0)),
        out_shape=jax.ShapeDtypeStruct((2 * n, 128), jnp.float32),
    )(x, W1)


def _mm2_body(h_ref, w_ref, o_ref):
    h = jnp.maximum(h_ref[...], 0.0)
    o_ref[...] = jnp.dot(h, w_ref[...], preferred_element_type=jnp.float32)


def _mm2(h, W2p, bm):
    n, hdim = h.shape
    cp = W2p.shape[1]
    return pl.pallas_call(
        _mm2_body,
        grid=(n // bm,),
        in_specs=[
            pl.BlockSpec((bm, hdim), lambda i: (i, 0)),
            pl.BlockSpec((hdim, cp), lambda i: (0, 0)),
        ],
        out_specs=pl.BlockSpec((bm, cp), lambda i: (i, 0)),
        out_shape=jax.ShapeDtypeStruct((n, cp), jnp.float32),
    )(h, W2p)


def _final_body(a0_ref, a1_ref, b_ref, o_ref):
    h = a0_ref[...] + a1_ref[...] + b_ref[...]
    m = jnp.max(h, axis=1, keepdims=True)
    lse = jnp.log(jnp.sum(jnp.exp(h - m), axis=1, keepdims=True))
    o_ref[...] = h - m - lse


def _final(a0, a1, b2p, bm):
    n, cp = a0.shape
    return pl.pallas_call(
        _final_body,
        grid=(n // bm,),
        in_specs=[
            pl.BlockSpec((bm, cp), lambda i: (i, 0)),
            pl.BlockSpec((bm, cp), lambda i: (i, 0)),
            pl.BlockSpec((1, cp), lambda i: (0, 0)),
        ],
        out_specs=pl.BlockSpec((bm, cp), lambda i: (i, 0)),
        out_shape=jax.ShapeDtypeStruct((n, cp), jnp.float32),
    )(a0, a1, b2p.reshape(1, cp))


def _agg1(sup_flat, src_all, dst3d, init1, ngrp, nr):
    nrow_sub = nr // NS
    grp_sub = ngrp // NS
    mesh = plsc.VectorSubcoreMesh(
        core_axis_name="c", subcore_axis_name="s", num_cores=NC,
        num_subcores=NS)

    @functools.partial(
        pl.kernel,
        out_type=jax.ShapeDtypeStruct((NC * nr, 128), jnp.float32),
        mesh=mesh,
        scratch_types=[
            pltpu.VMEM_SHARED((nr, 128), jnp.float32),
            pltpu.VMEM((8, LB), jnp.int32),
            pltpu.VMEM((8, LB), jnp.int32),
            pltpu.VMEM((LB, 128), jnp.float32),
            pltpu.SemaphoreType.DMA,
        ],
    )
    def k(sup_hbm, src_hbm, dst_hbm, init_hbm, out_hbm,
          agg_sh, src_v, dst_v, rows_v, sem):
        c = lax.axis_index("c")
        s = lax.axis_index("s")
        r0 = s * nrow_sub
        pltpu.sync_copy(init_hbm.at[pl.ds(c * nr + r0, nrow_sub)],
                        agg_sh.at[pl.ds(r0, nrow_sub)])
        plsc.subcore_barrier()
        base_s = c * ngrp + s * grp_sub
        base_d = s * grp_sub

        def body(j, carry):
            pltpu.sync_copy(src_hbm.at[base_s + j], src_v)
            pltpu.sync_copy(dst_hbm.at[base_d + j], dst_v)
            for i in range(8):
                pltpu.async_copy(sup_hbm.at[src_v.at[i]], rows_v, sem).wait()
                pltpu.sync_copy(rows_v, agg_sh.at[dst_v.at[i]], add=True)
            return carry

        lax.fori_loop(0, grp_sub, body, 0)
        plsc.subcore_barrier()
        pltpu.sync_copy(agg_sh.at[pl.ds(r0, nrow_sub)],
                        out_hbm.at[pl.ds(c * nr + r0, nrow_sub)])

    return k(sup_flat, src_all, dst3d, init1)


def _agg2(sup2, src3d, dst3d, zeros2, ngrp, nr, cp):
    nrow_sub = nr // NS
    grp_core = ngrp // NC
    grp_sub = grp_core // NS
    mesh = plsc.VectorSubcoreMesh(
        core_axis_name="c", subcore_axis_name="s", num_cores=NC,
        num_subcores=NS)

    @functools.partial(
        pl.kernel,
        out_type=jax.ShapeDtypeStruct((NC * nr, cp), jnp.float32),
        mesh=mesh,
        compiler_params=pltpu.CompilerParams(use_tc_tiling_on_sc=False),
        scratch_types=[
            pltpu.VMEM_SHARED((nr, cp), jnp.float32),
            pltpu.VMEM((8, LB), jnp.int32),
            pltpu.VMEM((8, LB), jnp.int32),
            pltpu.VMEM((LB, cp), jnp.float32),
            pltpu.SemaphoreType.DMA,
        ],
    )
    def k(sup_hbm, src_hbm, dst_hbm, zero_hbm, out_hbm,
          agg_sh, src_v, dst_v, rows_v, sem):
        c = lax.axis_index("c")
        s = lax.axis_index("s")
        r0 = s * nrow_sub
        pltpu.sync_copy(zero_hbm.at[pl.ds(r0, nrow_sub)],
                        agg_sh.at[pl.ds(r0, nrow_sub)])
        plsc.subcore_barrier()
        base = c * grp_core + s * grp_sub

        def body(j, carry):
            pltpu.sync_copy(src_hbm.at[base + j], src_v)
            pltpu.sync_copy(dst_hbm.at[base + j], dst_v)
            for i in range(8):
                pltpu.async_copy(sup_hbm.at[src_v.at[i]], rows_v, sem).wait()
                pltpu.sync_copy(rows_v, agg_sh.at[dst_v.at[i]], add=True)
            return carry

        lax.fori_loop(0, grp_sub, body, 0)
        plsc.subcore_barrier()
        pltpu.sync_copy(agg_sh.at[pl.ds(r0, nrow_sub)],
                        out_hbm.at[pl.ds(c * nr + r0, nrow_sub)])

    return k(sup2, src3d, dst3d, zeros2)


def kernel(x, edge_index, W1, b1, W2, b2):
    n, f = x.shape
    h = W1.shape[1]
    c_out = W2.shape[1]
    e = edge_index.shape[1]
    cp = 16
    ngrp = -(-e // (8 * LB * NC * NS)) * (NC * NS)
    ep = ngrp * 8 * LB
    nr = -(-(n + 1) // (8 * NS)) * (8 * NS)

    src = edge_index[0].astype(jnp.int32)
    dst = edge_index[1].astype(jnp.int32)
    pad = ep - e
    srcp = jnp.concatenate([src, jnp.zeros((pad,), jnp.int32)])
    dstp = jnp.concatenate([dst, jnp.full((pad,), n, jnp.int32)])
    src3d = srcp.reshape(ngrp, 8, LB)
    src_all = jnp.concatenate([src3d, src3d + n], axis=0)
    dst3d = dstp.reshape(ngrp, 8, LB)
    init1 = jnp.broadcast_to(b1.reshape(NC, 1, 128), (NC, nr, 128))
    init1 = init1.reshape(NC * nr, 128)
    zeros2 = jnp.zeros((nr, cp), jnp.float32)
    W2p = jnp.pad(W2, ((0, 0), (0, cp - c_out)))
    b2p = jnp.pad(b2, (0, cp - c_out), constant_values=-1e30)

    bm = 1000
    sup1 = _mm1(x, W1, bm)
    agg1 = _agg1(sup1, src_all, dst3d, init1, ngrp, nr)
    h1 = jnp.concatenate([agg1[:n], agg1[nr:nr + n]], axis=1)
    sup2 = _mm2(h1, W2p, bm)
    agg2 = _agg2(sup2, src3d, dst3d, zeros2, ngrp, nr, cp)
    out = _final(agg2[:n], agg2[nr:nr + n], b2p, bm)
    return out[:, :c_out]


# speedup vs baseline: 7.7971x; 1.4772x over previous
import functools

import jax
import jax.numpy as jnp
from jax import lax
from jax.experimental import pallas as pl
from jax.experimental.pallas import tpu as pltpu
from jax.experimental.pallas import tpu_sc as plsc

NC = 2
NS = 16
LB = 128


def _mm1_body(x_ref, w_ref, o_ref):
    c = pl.program_id(1)
    w = w_ref[:, pl.ds(c * 128, 128)]
    o_ref[...] = jnp.dot(x_ref[...], w, preferred_element_type=jnp.float32)


def _mm1(x, W1, bm):
    n, f = x.shape
    grid = (n // bm, 2)
    return pl.pallas_call(
        _mm1_body,
        grid=grid,
        in_specs=[
            pl.BlockSpec((bm, f), lambda i, c: (i, 0)),
            pl.BlockSpec((f, 256), lambda i, c: (0, 0)),
        ],
        out_specs=pl.BlockSpec((bm, 128), lambda i, c: (c * (n // bm) + i, 0)),
        out_shape=jax.ShapeDtypeStruct((2 * n, 128), jnp.float32),
    )(x, W1)


def _mm2_body(h_ref, w_ref, o_ref):
    h = jnp.maximum(h_ref[...], 0.0)
    o_ref[...] = jnp.dot(h, w_ref[...], preferred_element_type=jnp.float32)


def _mm2(h, W2p, bm):
    n, hdim = h.shape
    cp = W2p.shape[1]
    return pl.pallas_call(
        _mm2_body,
        grid=(n // bm,),
        in_specs=[
            pl.BlockSpec((bm, hdim), lambda i: (i, 0)),
            pl.BlockSpec((hdim, cp), lambda i: (0, 0)),
        ],
        out_specs=pl.BlockSpec((bm, cp), lambda i: (i, 0)),
        out_shape=jax.ShapeDtypeStruct((n, cp), jnp.float32),
    )(h, W2p)


def _final_body(a0_ref, a1_ref, b_ref, o_ref):
    h = a0_ref[...] + a1_ref[...] + b_ref[...]
    m = jnp.max(h, axis=1, keepdims=True)
    lse = jnp.log(jnp.sum(jnp.exp(h - m), axis=1, keepdims=True))
    o_ref[...] = h - m - lse


def _final(a0, a1, b2p, bm):
    n, cp = a0.shape
    return pl.pallas_call(
        _final_body,
        grid=(n // bm,),
        in_specs=[
            pl.BlockSpec((bm, cp), lambda i: (i, 0)),
            pl.BlockSpec((bm, cp), lambda i: (i, 0)),
            pl.BlockSpec((1, cp), lambda i: (0, 0)),
        ],
        out_specs=pl.BlockSpec((bm, cp), lambda i: (i, 0)),
        out_shape=jax.ShapeDtypeStruct((n, cp), jnp.float32),
    )(a0, a1, b2p.reshape(1, cp))


GB = 16


def _make_agg(*, d, nr, g_sub, src_stride, dst_stride, tc_tiling, gq):
    nrow_sub = nr // NS
    mesh = plsc.VectorSubcoreMesh(
        core_axis_name="c", subcore_axis_name="s", num_cores=NC,
        num_subcores=NS)
    scratch = [
        pltpu.VMEM_SHARED((nr, d), jnp.float32),
        pltpu.VMEM((gq, LB, d), jnp.float32),
        pltpu.VMEM((GB, LB), jnp.int32),
        pltpu.VMEM((GB, LB), jnp.int32),
        pltpu.VMEM((GB, LB), jnp.int32),
        pltpu.VMEM((GB, LB), jnp.int32),
    ] + [pltpu.SemaphoreType.DMA] * (gq + 2)

    def build(out_type):
        @functools.partial(
            pl.kernel, out_type=out_type, mesh=mesh,
            compiler_params=pltpu.CompilerParams(
                use_tc_tiling_on_sc=tc_tiling),
            scratch_types=scratch,
        )
        def k(sup, srcI, dstI, initI, out_hbm,
              agg_sh, ring, srcA, dstA, srcB, dstB, *sems):
            gsem = sems[:gq]
            isem = sems[gq:]
            c = lax.axis_index("c")
            s = lax.axis_index("s")
            r0 = s * nrow_sub
            pltpu.sync_copy(initI.at[pl.ds(c * nr + r0, nrow_sub)],
                            agg_sh.at[pl.ds(r0, nrow_sub)])
            plsc.subcore_barrier()
            sbase = c * src_stride + s * g_sub
            dbase = c * dst_stride + s * g_sub
            bufs = [(srcA, dstA, isem[0]), (srcB, dstB, isem[1])]

            def fetch_idx(g_off, parity):
                sb, db, sm = bufs[parity]
                pltpu.async_copy(srcI.at[sbase + g_off], sb, sm)
                pltpu.async_copy(dstI.at[dbase + g_off], db, sm)

            def process_group(g_off, parity, pre_off, pre_ok):
                sb, db, sm = bufs[parity]
                pltpu.make_async_copy(srcI.at[sbase], sb, sm).wait()
                pltpu.make_async_copy(dstI.at[dbase], db, sm).wait()
                ds = []
                for i in range(gq):
                    ds.append(pltpu.async_copy(
                        sup.at[sb.at[i]], ring.at[i % gq], gsem[i % gq]))
                for i in range(GB):
                    ds[i].wait()
                    if i + gq < GB:
                        ds.append(pltpu.async_copy(
                            sup.at[sb.at[i + gq]], ring.at[(i + gq) % gq],
                            gsem[(i + gq) % gq]))
                    pltpu.sync_copy(ring.at[i % gq], agg_sh.at[db.at[i]],
                                    add=True)

                @pl.when(pre_ok)
                def _():
                    fetch_idx(pre_off, parity)

            fetch_idx(0, 0)
            fetch_idx(1, 1)

            def body(u, carry):
                j = 2 * u
                process_group(j, 0, j + 2, j + 2 < g_sub)
                process_group(j + 1, 1, j + 3, j + 3 < g_sub)
                return carry

            lax.fori_loop(0, g_sub // 2, body, 0)
            plsc.subcore_barrier()
            pltpu.sync_copy(agg_sh.at[pl.ds(r0, nrow_sub)],
                            out_hbm.at[pl.ds(c * nr + r0, nrow_sub)])

        return k

    return build


def _agg1(sup_flat, src_all, dst3d, init1, grp_total, nr):
    build = _make_agg(d=128, nr=nr, g_sub=grp_total // NS,
                      src_stride=grp_total, dst_stride=0, tc_tiling=True, gq=2)
    k = build(jax.ShapeDtypeStruct((NC * nr, 128), jnp.float32))
    return k(sup_flat, src_all, dst3d, init1)


def _agg2(sup2, src3d, dst3d, zeros2, grp_total, nr, cp):
    build = _make_agg(d=cp, nr=nr, g_sub=grp_total // (NC * NS),
                      src_stride=grp_total // NC,
                      dst_stride=grp_total // NC, tc_tiling=False, gq=6)
    k = build(jax.ShapeDtypeStruct((NC * nr, cp), jnp.float32))
    return k(sup2, src3d, dst3d, zeros2)


def kernel(x, edge_index, W1, b1, W2, b2):
    n, f = x.shape
    h = W1.shape[1]
    c_out = W2.shape[1]
    e = edge_index.shape[1]
    cp = 16
    unit = LB * GB * NC * NS * 2
    ep = -(-e // unit) * unit
    grp_total = ep // (LB * GB)
    nr = -(-(n + 1) // (8 * NS)) * (8 * NS)

    src = edge_index[0].astype(jnp.int32)
    dst = edge_index[1].astype(jnp.int32)
    pad = ep - e
    srcp = jnp.concatenate([src, jnp.zeros((pad,), jnp.int32)])
    dstp = jnp.concatenate([dst, jnp.full((pad,), n, jnp.int32)])
    src3d = srcp.reshape(grp_total, GB, LB)
    src_all = jnp.concatenate([src3d, src3d + n], axis=0)
    dst3d = dstp.reshape(grp_total, GB, LB)
    init1 = jnp.broadcast_to(b1.reshape(NC, 1, 128), (NC, nr, 128))
    init1 = init1.reshape(NC * nr, 128)
    zeros2 = jnp.zeros((NC * nr, cp), jnp.float32)
    W2p = jnp.pad(W2, ((0, 0), (0, cp - c_out)))
    b2p = jnp.pad(b2, (0, cp - c_out), constant_values=-1e30)

    bm = 1000
    sup1 = _mm1(x, W1, bm)
    agg1 = _agg1(sup1, src_all, dst3d, init1, grp_total, nr)
    h1 = jnp.concatenate([agg1[:n], agg1[nr:nr + n]], axis=1)
    sup2 = _mm2(h1, W2p, bm)
    agg2 = _agg2(sup2, src3d, dst3d, zeros2, grp_total, nr, cp)
    out = _final(agg2[:n], agg2[nr:nr + n], b2p, bm)
    return out[:, :c_out]


# speedup vs baseline: 7.8840x; 1.0111x over previous
import functools

import jax
import jax.numpy as jnp
from jax import lax
from jax.experimental import pallas as pl
from jax.experimental.pallas import tpu as pltpu
from jax.experimental.pallas import tpu_sc as plsc

NC = 2
NS = 16
LB = 128


def _mm1_body(x_ref, w_ref, o_ref):
    c = pl.program_id(1)
    w = w_ref[:, pl.ds(c * 128, 128)]
    o_ref[...] = jnp.dot(x_ref[...], w, preferred_element_type=jnp.float32)


def _mm1(x, W1, bm):
    n, f = x.shape
    grid = (n // bm, 2)
    return pl.pallas_call(
        _mm1_body,
        grid=grid,
        in_specs=[
            pl.BlockSpec((bm, f), lambda i, c: (i, 0)),
            pl.BlockSpec((f, 256), lambda i, c: (0, 0)),
        ],
        out_specs=pl.BlockSpec((bm, 128), lambda i, c: (c * (n // bm) + i, 0)),
        out_shape=jax.ShapeDtypeStruct((2 * n, 128), jnp.float32),
    )(x, W1)


def _mm2_body(h_ref, w_ref, o_ref):
    h = jnp.maximum(h_ref[...], 0.0)
    o_ref[...] = jnp.dot(h, w_ref[...], preferred_element_type=jnp.float32)


def _mm2(h, W2p, bm):
    n, hdim = h.shape
    cp = W2p.shape[1]
    return pl.pallas_call(
        _mm2_body,
        grid=(n // bm,),
        in_specs=[
            pl.BlockSpec((bm, hdim), lambda i: (i, 0)),
            pl.BlockSpec((hdim, cp), lambda i: (0, 0)),
        ],
        out_specs=pl.BlockSpec((bm, cp), lambda i: (i, 0)),
        out_shape=jax.ShapeDtypeStruct((n, cp), jnp.float32),
    )(h, W2p)


def _final_body(a0_ref, a1_ref, b_ref, o_ref):
    h = a0_ref[...] + a1_ref[...] + b_ref[...]
    m = jnp.max(h, axis=1, keepdims=True)
    lse = jnp.log(jnp.sum(jnp.exp(h - m), axis=1, keepdims=True))
    o_ref[...] = h - m - lse


def _final(a0, a1, b2p, bm):
    n, cp = a0.shape
    return pl.pallas_call(
        _final_body,
        grid=(n // bm,),
        in_specs=[
            pl.BlockSpec((bm, cp), lambda i: (i, 0)),
            pl.BlockSpec((bm, cp), lambda i: (i, 0)),
            pl.BlockSpec((1, cp), lambda i: (0, 0)),
        ],
        out_specs=pl.BlockSpec((bm, cp), lambda i: (i, 0)),
        out_shape=jax.ShapeDtypeStruct((n, cp), jnp.float32),
    )(a0, a1, b2p.reshape(1, cp))


GB = 16


def _make_agg(*, d, nr, g_sub, src_stride, dst_stride, tc_tiling, gq, gb=GB):
    nrow_sub = nr // NS
    mesh = plsc.VectorSubcoreMesh(
        core_axis_name="c", subcore_axis_name="s", num_cores=NC,
        num_subcores=NS)
    scratch = [
        pltpu.VMEM_SHARED((nr, d), jnp.float32),
        pltpu.VMEM((gq, LB, d), jnp.float32),
        pltpu.VMEM((gb, LB), jnp.int32),
        pltpu.VMEM((gb, LB), jnp.int32),
        pltpu.VMEM((gb, LB), jnp.int32),
        pltpu.VMEM((gb, LB), jnp.int32),
    ] + [pltpu.SemaphoreType.DMA] * (gq + 2)

    def build(out_type):
        @functools.partial(
            pl.kernel, out_type=out_type, mesh=mesh,
            compiler_params=pltpu.CompilerParams(
                use_tc_tiling_on_sc=tc_tiling),
            scratch_types=scratch,
        )
        def k(sup, srcI, dstI, initI, out_hbm,
              agg_sh, ring, srcA, dstA, srcB, dstB, *sems):
            gsem = sems[:gq]
            isem = sems[gq:]
            c = lax.axis_index("c")
            s = lax.axis_index("s")
            r0 = s * nrow_sub
            pltpu.sync_copy(initI.at[pl.ds(c * nr + r0, nrow_sub)],
                            agg_sh.at[pl.ds(r0, nrow_sub)])
            plsc.subcore_barrier()
            sbase = c * src_stride + s * g_sub
            dbase = c * dst_stride + s * g_sub
            bufs = [(srcA, dstA, isem[0]), (srcB, dstB, isem[1])]

            def fetch_idx(g_off, parity):
                sb, db, sm = bufs[parity]
                pltpu.async_copy(srcI.at[sbase + g_off], sb, sm)
                pltpu.async_copy(dstI.at[dbase + g_off], db, sm)

            def process_group(g_off, parity, pre_off, pre_ok):
                sb, db, sm = bufs[parity]
                pltpu.make_async_copy(srcI.at[sbase], sb, sm).wait()
                pltpu.make_async_copy(dstI.at[dbase], db, sm).wait()
                ds = []
                for i in range(gq):
                    ds.append(pltpu.async_copy(
                        sup.at[sb.at[i]], ring.at[i % gq], gsem[i % gq]))
                for i in range(gb):
                    ds[i].wait()
                    if i + gq < gb:
                        ds.append(pltpu.async_copy(
                            sup.at[sb.at[i + gq]], ring.at[(i + gq) % gq],
                            gsem[(i + gq) % gq]))
                    pltpu.sync_copy(ring.at[i % gq], agg_sh.at[db.at[i]],
                                    add=True)

                @pl.when(pre_ok)
                def _():
                    fetch_idx(pre_off, parity)

            fetch_idx(0, 0)
            fetch_idx(1, 1)

            def body(u, carry):
                j = 2 * u
                process_group(j, 0, j + 2, j + 2 < g_sub)
                process_group(j + 1, 1, j + 3, j + 3 < g_sub)
                return carry

            lax.fori_loop(0, g_sub // 2, body, 0)
            plsc.subcore_barrier()
            pltpu.sync_copy(agg_sh.at[pl.ds(r0, nrow_sub)],
                            out_hbm.at[pl.ds(c * nr + r0, nrow_sub)])

        return k

    return build


def _agg1(sup_flat, src_all, dst3d, init1, grp32, nr):
    build = _make_agg(d=128, nr=nr, g_sub=grp32 // NS,
                      src_stride=grp32, dst_stride=0, tc_tiling=True,
                      gq=2, gb=32)
    k = build(jax.ShapeDtypeStruct((NC * nr, 128), jnp.float32))
    return k(sup_flat, src_all, dst3d, init1)


def _agg2(sup2, src3d, dst3d, zeros2, grp_total, nr, cp):
    build = _make_agg(d=cp, nr=nr, g_sub=grp_total // (NC * NS),
                      src_stride=grp_total // NC,
                      dst_stride=grp_total // NC, tc_tiling=False, gq=6)
    k = build(jax.ShapeDtypeStruct((NC * nr, cp), jnp.float32))
    return k(sup2, src3d, dst3d, zeros2)


def kernel(x, edge_index, W1, b1, W2, b2):
    n, f = x.shape
    h = W1.shape[1]
    c_out = W2.shape[1]
    e = edge_index.shape[1]
    cp = 16
    unit = LB * GB * NC * NS * 2
    ep = -(-e // unit) * unit
    grp_total = ep // (LB * GB)
    nr = -(-(n + 1) // (8 * NS)) * (8 * NS)

    src = edge_index[0].astype(jnp.int32)
    dst = edge_index[1].astype(jnp.int32)
    pad = ep - e
    srcp = jnp.concatenate([src, jnp.zeros((pad,), jnp.int32)])
    dstp = jnp.concatenate([dst, jnp.full((pad,), n, jnp.int32)])
    src3d = srcp.reshape(grp_total, GB, LB)
    dst3d = dstp.reshape(grp_total, GB, LB)
    grp32 = ep // (LB * 32)
    src32 = srcp.reshape(grp32, 32, LB)
    src_all = jnp.concatenate([src32, src32 + n], axis=0)
    dst32 = dstp.reshape(grp32, 32, LB)
    init1 = jnp.broadcast_to(b1.reshape(NC, 1, 128), (NC, nr, 128))
    init1 = init1.reshape(NC * nr, 128)
    zeros2 = jnp.zeros((NC * nr, cp), jnp.float32)
    W2p = jnp.pad(W2, ((0, 0), (0, cp - c_out)))
    b2p = jnp.pad(b2, (0, cp - c_out), constant_values=-1e30)

    bm = 1000
    sup1 = _mm1(x, W1, bm)
    agg1 = _agg1(sup1, src_all, dst32, init1, grp32, nr)
    h1 = jnp.concatenate([agg1[:n], agg1[nr:nr + n]], axis=1)
    sup2 = _mm2(h1, W2p, bm)
    agg2 = _agg2(sup2, src3d, dst3d, zeros2, grp_total, nr, cp)
    out = _final(agg2[:n], agg2[nr:nr + n], b2p, bm)
    return out[:, :c_out]
